# baseline (speedup 1.0000x reference)
import functools

import jax
import jax.numpy as jnp
from jax import lax
from jax.experimental import pallas as pl
from jax.experimental.pallas import tpu as pltpu
from jax.experimental.pallas import tpu_sc as plsc

_NC, _NS = 2, 16
_NW = _NC * _NS
_NCH = 24
_G = 8


def _streamed_copy(src, idx_v, out, base, ng, bufs, gsems, ssems):
    for g in range(ng):
        slot = g % 2
        if g >= 2:
            pltpu.make_async_copy(
                bufs[slot], out.at[pl.ds(base + (g - 2) * _G, _G)], ssems[slot]
            ).wait()
        pltpu.async_copy(
            src.at[idx_v.at[pl.ds(g * _G, _G)]], bufs[slot], gsems[slot]
        ).wait()
        pltpu.async_copy(bufs[slot], out.at[pl.ds(base + g * _G, _G)], ssems[slot])
    for g in range(max(ng - 2, 0), ng):
        slot = g % 2
        pltpu.make_async_copy(
            bufs[slot], out.at[pl.ds(base + g * _G, _G)], ssems[slot]
        ).wait()


@functools.lru_cache(maxsize=None)
def _make_gather(rows_i, rows_t, fc):
    rpw_i = rows_i // _NW
    rpw_t = rows_t // _NW
    ng_i = rpw_i // _G
    ng_t = rpw_t // _G
    assert rpw_i % _G == 0 and rpw_t % _G == 0

    mesh = plsc.VectorSubcoreMesh(
        core_axis_name="c", subcore_axis_name="s",
        num_cores=_NC, num_subcores=_NS,
    )

    @functools.partial(
        pl.kernel,
        out_type=(
            jax.ShapeDtypeStruct((rows_i, fc), jnp.float32),
            jax.ShapeDtypeStruct((rows_t, fc), jnp.float32),
        ),
        mesh=mesh,
        scratch_types=[
            pltpu.VMEM((rpw_i,), jnp.int32),
            pltpu.VMEM((rpw_t,), jnp.int32),
            pltpu.VMEM((_G, fc), jnp.float32),
            pltpu.VMEM((_G, fc), jnp.float32),
            pltpu.SemaphoreType.DMA,
            pltpu.SemaphoreType.DMA,
            pltpu.SemaphoreType.DMA,
            pltpu.SemaphoreType.DMA,
        ],
    )
    def gather_kernel(src, idx_i, idx_t, out_i, out_t,
                      idxi_v, idxt_v, buf0, buf1, sg0, sg1, ss0, ss1):
        w = lax.axis_index("s") * _NC + lax.axis_index("c")
        pltpu.sync_copy(idx_i.at[pl.ds(w * rpw_i, rpw_i)], idxi_v)
        pltpu.sync_copy(idx_t.at[pl.ds(w * rpw_t, rpw_t)], idxt_v)
        bufs, gsems, ssems = (buf0, buf1), (sg0, sg1), (ss0, ss1)
        _streamed_copy(src, idxi_v, out_i, w * rpw_i, ng_i, bufs, gsems, ssems)
        _streamed_copy(src, idxt_v, out_t, w * rpw_t, ng_t, bufs, gsems, ssems)

    return gather_kernel


def kernel(image, context_indices, target_indices):
    B, V, C, H, W = image.shape
    ni = context_indices.shape[1]
    nt = target_indices.shape[1]
    F = C * H * W
    assert F % _NCH == 0
    fc = F // _NCH
    rows_i = B * ni * _NCH
    rows_t = B * nt * _NCH
    assert rows_i % (_NW * _G) == 0 and rows_t % (_NW * _G) == 0

    src = image.reshape(B * V * _NCH, fc)
    bi = jnp.arange(B, dtype=jnp.int32)[:, None]
    chunk = jnp.arange(_NCH, dtype=jnp.int32)[None, :]
    idx_i = ((bi * V + context_indices).reshape(-1)[:, None] * _NCH + chunk).reshape(-1)
    idx_t = ((bi * V + target_indices).reshape(-1)[:, None] * _NCH + chunk).reshape(-1)

    out_i, out_t = _make_gather(rows_i, rows_t, fc)(src, idx_i, idx_t)
    return (out_i.reshape(B, ni, C, H, W),
            out_t.reshape(B, nt, C, H, W),
            context_indices, target_indices)


# speedup vs baseline: 1.0104x; 1.0104x over previous
import functools

import jax
import jax.numpy as jnp
from jax import lax
from jax.experimental import pallas as pl
from jax.experimental.pallas import tpu as pltpu
from jax.experimental.pallas import tpu_sc as plsc

_NC, _NS = 2, 16
_NW = _NC * _NS
_NCH = 56
_G = 8
_NB = 5
_L = 2


@functools.lru_cache(maxsize=None)
def _make_gather(rows_i, rows_t, fc):
    rpw_i = rows_i // _NW
    rpw_t = rows_t // _NW
    ng_i = rpw_i // _G
    ng_t = rpw_t // _G
    assert rpw_i % _G == 0 and rpw_t % _G == 0

    mesh = plsc.VectorSubcoreMesh(
        core_axis_name="c", subcore_axis_name="s",
        num_cores=_NC, num_subcores=_NS,
    )

    @functools.partial(
        pl.kernel,
        out_type=(
            jax.ShapeDtypeStruct((rows_i, fc), jnp.float32),
            jax.ShapeDtypeStruct((rows_t, fc), jnp.float32),
        ),
        mesh=mesh,
        scratch_types=[
            pltpu.VMEM((rpw_i,), jnp.int32),
            pltpu.VMEM((rpw_t,), jnp.int32),
        ] + [pltpu.VMEM((_G, fc), jnp.float32) for _ in range(_NB)]
          + [pltpu.SemaphoreType.DMA for _ in range(2 * _NB)],
    )
    def gather_kernel(src, idx_i, idx_t, out_i, out_t, idxi_v, idxt_v, *rest):
        bufs, gsems, ssems = rest[:_NB], rest[_NB:2 * _NB], rest[2 * _NB:]
        w = lax.axis_index("s") * _NC + lax.axis_index("c")
        pltpu.sync_copy(idx_i.at[pl.ds(w * rpw_i, rpw_i)], idxi_v)
        pltpu.sync_copy(idx_t.at[pl.ds(w * rpw_t, rpw_t)], idxt_v)

        items = [(idxi_v, g * _G, out_i, w * rpw_i + g * _G) for g in range(ng_i)]
        items += [(idxt_v, g * _G, out_t, w * rpw_t + g * _G) for g in range(ng_t)]
        n = len(items)

        def gat(k):
            idx_v, ioff, _, _ = items[k]
            return pltpu.make_async_copy(
                src.at[idx_v.at[pl.ds(ioff, _G)]], bufs[k % _NB], gsems[k % _NB])

        def sca(k):
            _, _, out, obase = items[k]
            return pltpu.make_async_copy(
                bufs[k % _NB], out.at[pl.ds(obase, _G)], ssems[k % _NB])

        for k in range(min(_L, n)):
            gat(k).start()
        for i in range(n):
            k = i + _L
            if k < n:
                if k >= _NB:
                    sca(k - _NB).wait()
                gat(k).start()
            gat(i).wait()
            sca(i).start()
        for j in range(max(0, n - _NB + _L), n):
            sca(j).wait()

    return gather_kernel


def kernel(image, context_indices, target_indices):
    B, V, C, H, W = image.shape
    ni = context_indices.shape[1]
    nt = target_indices.shape[1]
    F = C * H * W
    assert F % _NCH == 0
    fc = F // _NCH
    rows_i = B * ni * _NCH
    rows_t = B * nt * _NCH
    assert rows_i % (_NW * _G) == 0 and rows_t % (_NW * _G) == 0

    src = image.reshape(B * V * _NCH, fc)
    bi = jnp.arange(B, dtype=jnp.int32)[:, None]
    chunk = jnp.arange(_NCH, dtype=jnp.int32)[None, :]
    idx_i = ((bi * V + context_indices).reshape(-1)[:, None] * _NCH + chunk).reshape(-1)
    idx_t = ((bi * V + target_indices).reshape(-1)[:, None] * _NCH + chunk).reshape(-1)

    out_i, out_t = _make_gather(rows_i, rows_t, fc)(src, idx_i, idx_t)
    return (out_i.reshape(B, ni, C, H, W),
            out_t.reshape(B, nt, C, H, W),
            context_indices, target_indices)


# speedup vs baseline: 3.4899x; 3.4539x over previous
import functools

import jax
import jax.numpy as jnp
from jax import lax
from jax.experimental import pallas as pl
from jax.experimental.pallas import tpu as pltpu
from jax.experimental.pallas import tpu_sc as plsc

_NC, _NS = 2, 16
_NW = _NC * _NS


@functools.lru_cache(maxsize=None)
def _make_split(R, Ri, Rt, H, W):
    bpw_i = Ri // _NW
    bpw_t = Rt // _NW
    bpw = bpw_i + bpw_t
    tab_w = (bpw + 15) // 16 * 16
    assert Ri % _NW == 0 and Rt % _NW == 0 and bpw <= 32

    mesh = plsc.VectorSubcoreMesh(
        core_axis_name="c", subcore_axis_name="s",
        num_cores=_NC, num_subcores=_NS,
    )

    @functools.partial(
        pl.kernel,
        out_type=(
            jax.ShapeDtypeStruct((Ri, H, W), jnp.float32),
            jax.ShapeDtypeStruct((Rt, H, W), jnp.float32),
        ),
        mesh=mesh,
        scratch_types=[
            pltpu.VMEM((tab_w,), jnp.int32),
            pltpu.VMEM((H, W), jnp.float32),
            pltpu.VMEM((H, W), jnp.float32),
            pltpu.SemaphoreType.DMA,
            pltpu.SemaphoreType.DMA,
            pltpu.SemaphoreType.DMA,
            pltpu.SemaphoreType.DMA,
        ],
    )
    def split_kernel(img, tab, out_i, out_t, tab_v, buf0, buf1, sg0, sg1, ss0, ss1):
        w = lax.axis_index("s") * _NC + lax.axis_index("c")
        pltpu.sync_copy(tab.at[w], tab_v)
        svs = [tab_v[pl.ds(16 * i, 16)] for i in range(tab_w // 16)]
        bufs, gsems, ssems = (buf0, buf1), (sg0, sg1), (ss0, ss1)

        def gat(k):
            sv, lane = svs[k // 16], k % 16
            src = lax.squeeze(lax.slice(sv, (lane,), (lane + 1,)), (0,))
            return pltpu.make_async_copy(
                img.at[src], bufs[k % 2], gsems[k % 2])

        def sca(k):
            if k < bpw_i:
                dst_ref, dst = out_i, w * bpw_i + k
            else:
                dst_ref, dst = out_t, w * bpw_t + (k - bpw_i)
            return pltpu.make_async_copy(
                bufs[k % 2], dst_ref.at[dst], ssems[k % 2])

        gat(0).start()
        for k in range(bpw):
            if k + 1 < bpw:
                if k >= 1:
                    sca(k - 1).wait()
                gat(k + 1).start()
            gat(k).wait()
            sca(k).start()
        for j in range(max(0, bpw - 2), bpw):
            sca(j).wait()

    return split_kernel


def kernel(image, context_indices, target_indices):
    B, V, C, H, W = image.shape
    ni = context_indices.shape[1]
    nt = target_indices.shape[1]
    Ri, Rt = B * ni * C, B * nt * C

    img3 = image.reshape(B * V * C, H, W)
    bi = jnp.arange(B, dtype=jnp.int32)[:, None]
    ch = jnp.arange(C, dtype=jnp.int32)[None, None, :]
    src_i = (((bi * V + context_indices) * C)[..., None] + ch).reshape(_NW, -1)
    src_t = (((bi * V + target_indices) * C)[..., None] + ch).reshape(_NW, -1)
    bpw = (Ri + Rt) // _NW
    pad = (bpw + 15) // 16 * 16 - bpw
    tab = jnp.concatenate(
        [src_i, src_t, jnp.zeros((_NW, pad), jnp.int32)], axis=1)

    out_i, out_t = _make_split(B * V * C, Ri, Rt, H, W)(img3, tab)
    return (out_i.reshape(B, ni, C, H, W),
            out_t.reshape(B, nt, C, H, W),
            context_indices, target_indices)
